# baseline (speedup 1.0000x reference)
import functools

import jax
import jax.numpy as jnp
from jax import lax
from jax.experimental import pallas as pl
from jax.experimental.pallas import tpu as pltpu
from jax.experimental.pallas import tpu_sc as plsc

_NC = 2
_NS = 16
_NW = _NC * _NS
_LANES = 16
_CH = 16
_UNROLL = 8


def kernel(x, pos_embedding):
    b, s, d = x.shape
    rows_per_w = s // _NW
    n_chunks = rows_per_w // _CH
    n_steps = n_chunks * b
    groups = d // _LANES
    mesh = plsc.VectorSubcoreMesh(core_axis_name="c", subcore_axis_name="s")

    @functools.partial(
        pl.kernel,
        mesh=mesh,
        out_type=jax.ShapeDtypeStruct((b, s, d), jnp.float32),
        scratch_types=[
            pltpu.VMEM((2, _CH, d), jnp.float32),
            pltpu.VMEM((2, _CH, d), jnp.float32),
            pltpu.SemaphoreType.DMA((2,)),
            pltpu.SemaphoreType.DMA((2,)),
            pltpu.SemaphoreType.DMA((2,)),
        ],
    )
    def k(x_hbm, pos_hbm, out_hbm, pos_v, x_v, sem_in, sem_out, sem_pos):
        wid = lax.axis_index("s") * _NC + lax.axis_index("c")
        row_base = wid * rows_per_w

        def chunk_rows(c):
            return row_base + c * _CH

        pltpu.async_copy(pos_hbm.at[pl.ds(chunk_rows(0), _CH)], pos_v.at[0],
                         sem_pos.at[0])
        pltpu.async_copy(x_hbm.at[0, pl.ds(chunk_rows(0), _CH)], x_v.at[0],
                         sem_in.at[0])

        def step(t):
            xb = t % 2
            c, bi = t // b, t % b
            pb = c % 2
            r0 = chunk_rows(c)
            if bi == 0 and c + 1 < n_chunks:
                pltpu.async_copy(
                    pos_hbm.at[pl.ds(chunk_rows(c + 1), _CH)],
                    pos_v.at[(pb + 1) % 2], sem_pos.at[(pb + 1) % 2])
            t1 = t + 1
            if t1 < n_steps:
                c1, b1 = t1 // b, t1 % b
                if t1 >= 2:
                    t2 = t1 - 2
                    pltpu.make_async_copy(
                        x_v.at[(xb + 1) % 2],
                        out_hbm.at[t2 % b, pl.ds(chunk_rows(t2 // b), _CH)],
                        sem_out.at[(xb + 1) % 2]).wait()
                pltpu.async_copy(
                    x_hbm.at[b1, pl.ds(chunk_rows(c1), _CH)],
                    x_v.at[(xb + 1) % 2], sem_in.at[(xb + 1) % 2])
            pltpu.make_async_copy(
                x_hbm.at[bi, pl.ds(r0, _CH)], x_v.at[xb], sem_in.at[xb]).wait()
            if bi == 0:
                pltpu.make_async_copy(
                    pos_hbm.at[pl.ds(r0, _CH)], pos_v.at[pb],
                    sem_pos.at[pb]).wait()

            def row_body(r, _):
                def col_body(j, _):
                    for u in range(_UNROLL):
                        o = (j * _UNROLL + u) * _LANES
                        x_v[xb, r, pl.ds(o, _LANES)] = (
                            x_v[xb, r, pl.ds(o, _LANES)]
                            + pos_v[pb, r, pl.ds(o, _LANES)])
                    return 0

                lax.fori_loop(0, groups // _UNROLL, col_body, 0)
                return 0

            lax.fori_loop(0, _CH, row_body, 0)
            pltpu.async_copy(x_v.at[xb], out_hbm.at[bi, pl.ds(r0, _CH)],
                             sem_out.at[xb])

        for t in range(n_steps):
            step(t)
        for t in (n_steps - 2, n_steps - 1):
            pltpu.make_async_copy(
                x_v.at[t % 2],
                out_hbm.at[t % b, pl.ds(chunk_rows(t // b), _CH)],
                sem_out.at[t % 2]).wait()

    return k(x, pos_embedding)


# speedup vs baseline: 1.8245x; 1.8245x over previous
import functools

import jax
import jax.numpy as jnp
from jax import lax
from jax.experimental import pallas as pl
from jax.experimental.pallas import tpu as pltpu
from jax.experimental.pallas import tpu_sc as plsc

_NC = 2
_NS = 16
_NW = _NC * _NS
_LANES = 16
_CH = 16
_UNROLL = 8


def kernel(x, pos_embedding):
    b, s, d = x.shape
    rows_per_w = s // _NW
    n_chunks = rows_per_w // _CH
    n_steps = n_chunks * b
    steps_per_iter = 2 * b
    n_iters = n_steps // steps_per_iter
    groups = d // _LANES
    mesh = plsc.VectorSubcoreMesh(core_axis_name="c", subcore_axis_name="s")

    @functools.partial(
        pl.kernel,
        mesh=mesh,
        out_type=jax.ShapeDtypeStruct((b, s, d), jnp.float32),
        scratch_types=[
            pltpu.VMEM((2, _CH, d), jnp.float32),
            pltpu.VMEM((2, _CH, d), jnp.float32),
            pltpu.SemaphoreType.DMA((2,)),
            pltpu.SemaphoreType.DMA((2,)),
            pltpu.SemaphoreType.DMA((2,)),
        ],
    )
    def k(x_hbm, pos_hbm, out_hbm, pos_v, x_v, sem_in, sem_out, sem_pos):
        wid = lax.axis_index("s") * _NC + lax.axis_index("c")
        row_base = wid * rows_per_w

        pltpu.async_copy(pos_hbm.at[pl.ds(row_base, _CH)], pos_v.at[0],
                         sem_pos.at[0])
        pltpu.async_copy(x_hbm.at[0, pl.ds(row_base, _CH)], x_v.at[0],
                         sem_in.at[0])

        def iter_body(cp, _):
            c0 = cp * 2
            r0 = row_base + c0 * _CH

            for u in range(steps_per_iter):
                k_, bi = u // b, u % b
                xb = u % 2
                pb = k_
                rows = r0 + k_ * _CH

                if u == 0:
                    pltpu.async_copy(
                        pos_hbm.at[pl.ds(rows + _CH, _CH)],
                        pos_v.at[1], sem_pos.at[1])
                elif u == b:
                    pltpu.async_copy(
                        pos_hbm.at[pl.ds(rows + _CH, _CH)],
                        pos_v.at[0], sem_pos.at[0])

                ob = (xb + 1) % 2
                if u == 0:
                    @pl.when(cp > 0)
                    def _():
                        pltpu.make_async_copy(
                            x_v.at[ob],
                            out_hbm.at[b - 1, pl.ds(r0 - _CH, _CH)],
                            sem_out.at[ob]).wait()
                    pltpu.async_copy(
                        x_hbm.at[1, pl.ds(rows, _CH)],
                        x_v.at[ob], sem_in.at[ob])
                else:
                    tp = u - 1
                    pltpu.make_async_copy(
                        x_v.at[ob],
                        out_hbm.at[tp % b, pl.ds(r0 + (tp // b) * _CH, _CH)],
                        sem_out.at[ob]).wait()
                    tn = u + 1
                    if tn < steps_per_iter:
                        pltpu.async_copy(
                            x_hbm.at[tn % b, pl.ds(r0 + (tn // b) * _CH, _CH)],
                            x_v.at[ob], sem_in.at[ob])
                    else:
                        @pl.when(cp + 1 < n_iters)
                        def _():
                            pltpu.async_copy(
                                x_hbm.at[0, pl.ds(r0 + 2 * _CH, _CH)],
                                x_v.at[ob], sem_in.at[ob])

                pltpu.make_async_copy(
                    x_hbm.at[bi, pl.ds(rows, _CH)], x_v.at[xb],
                    sem_in.at[xb]).wait()
                if bi == 0:
                    pltpu.make_async_copy(
                        pos_hbm.at[pl.ds(rows, _CH)], pos_v.at[pb],
                        sem_pos.at[pb]).wait()

                def row_body(r, _):
                    def col_body(j, _):
                        for v in range(_UNROLL):
                            o = (j * _UNROLL + v) * _LANES
                            x_v[xb, r, pl.ds(o, _LANES)] = (
                                x_v[xb, r, pl.ds(o, _LANES)]
                                + pos_v[pb, r, pl.ds(o, _LANES)])
                        return 0

                    lax.fori_loop(0, groups // _UNROLL, col_body, 0,
                                  unroll=True)
                    return 0

                lax.fori_loop(0, _CH, row_body, 0)

                pltpu.async_copy(x_v.at[xb], out_hbm.at[bi, pl.ds(rows, _CH)],
                                 sem_out.at[xb])
            return 0

        lax.fori_loop(0, n_iters, iter_body, 0)

        last_rows = row_base + (n_chunks - 1) * _CH
        pltpu.make_async_copy(
            x_v.at[1], out_hbm.at[b - 1, pl.ds(last_rows, _CH)],
            sem_out.at[1]).wait()
        pltpu.make_async_copy(
            pos_hbm.at[pl.ds(row_base, _CH)], pos_v.at[0],
            sem_pos.at[0]).wait()

    return k(x, pos_embedding)
